# baseline (speedup 1.0000x reference)
import functools
import math

import jax
import jax.numpy as jnp
import numpy as np
from jax import lax
from jax.experimental import pallas as pl
from jax.experimental.pallas import tpu as pltpu
from jax.experimental.pallas import tpu_sc as plsc

_D_MODEL = 1024
_K = 8
_D_HASH = 128
_N_BUCKETS = 32768
_N_TYPES = 6
_PRIMES = (10007, 10009, 10037, 10039, 10061, 10067, 10069, 10079)
_OFFSETS = (11, 29, 47, 71, 101, 131, 149, 173)


def _pos_encoding(T, d):
    position = np.arange(T, dtype=np.float32)[:, None]
    div_term = np.exp(np.arange(0, d, 2, dtype=np.float32) * (-math.log(10000.0) / d))
    pe = np.zeros((T, d), dtype=np.float32)
    pe[:, 0::2] = np.sin(position * div_term)
    pe[:, 1::2] = np.cos(position * div_term)
    return jnp.asarray(pe)


@functools.lru_cache(maxsize=None)
def _make_sc_gather(n_tok):
    info = plsc.get_sparse_core_info()
    nc, ns = info.num_cores, info.num_subcores
    nw = nc * ns
    tok_per_w = n_tok // nw
    n_rows = n_tok * _K

    mesh = plsc.VectorSubcoreMesh(core_axis_name="c", subcore_axis_name="s")

    n_gg = tok_per_w // 128
    ng = _K * n_gg
    nb = 6
    fire_ahead = 2

    @functools.partial(
        pl.kernel,
        mesh=mesh,
        out_type=jax.ShapeDtypeStruct((n_rows, _D_HASH), jnp.float32),
        scratch_types=[
            pltpu.VMEM((tok_per_w,), jnp.int32),
            pltpu.VMEM((ng, 128), jnp.int32),
            pltpu.VMEM((nb * 128, _D_HASH), jnp.float32),
            pltpu.SemaphoreType.DMA,
            pltpu.SemaphoreType.DMA,
        ],
    )
    def sc_gather(tok_hbm, table_hbm, out_hbm, tok_v, idx_v, rows_v, gsem, osem):
        wid = lax.axis_index("s") * nc + lax.axis_index("c")
        pltpu.sync_copy(tok_hbm.at[pl.ds(wid * tok_per_w, tok_per_w)], tok_v)

        for k in range(_K):
            for gg in range(n_gg):

                def cbody(j, carry, k=k, gg=gg):
                    tokv = tok_v[pl.ds(gg * 128 + j * 16, 16)]
                    hk = ((tokv * _PRIMES[k] + _OFFSETS[k]) & (_N_BUCKETS - 1)) + k * _N_BUCKETS
                    idx_v[k * n_gg + gg, pl.ds(j * 16, 16)] = hk
                    return carry

                lax.fori_loop(0, 8, cbody, 0)

        wbase = wid * tok_per_w

        gg_shift = n_gg.bit_length() - 1

        def obase(g):
            return (g >> gg_shift) * n_tok + wbase + (g & (n_gg - 1)) * 128

        def slot_ref(g):
            return rows_v.at[pl.ds(lax.rem(g, nb) * 128, 128)]

        def body(g, carry):
            @pl.when(g < ng)
            def _():
                @pl.when(g >= nb)
                def _():
                    pltpu.make_async_copy(
                        slot_ref(g), out_hbm.at[pl.ds(obase(g - nb), 128)], osem
                    ).wait()

                pltpu.async_copy(table_hbm.at[idx_v.at[g]], slot_ref(g), gsem)

            h = g - fire_ahead

            @pl.when(h >= 0)
            def _():
                pltpu.make_async_copy(
                    table_hbm.at[idx_v.at[h]], slot_ref(h), gsem
                ).wait()
                pltpu.async_copy(slot_ref(h), out_hbm.at[pl.ds(obase(h), 128)], osem)

            return carry

        lax.fori_loop(0, ng + fire_ahead, body, 0)

        def drain(i, carry):
            g = ng - nb + i
            pltpu.make_async_copy(
                slot_ref(g), out_hbm.at[pl.ds(obase(g), 128)], osem
            ).wait()
            return carry

        lax.fori_loop(0, nb, drain, 0)

    return sc_gather


_BT = 512


def _post_body(pool, B):
    def body(x_ref, pe_ref, tid_ref, tt_ref, gate_ref, w_ref, o_ref):
        pe = pe_ref[...].astype(jnp.float32)
        for b in range(B):
            x = jnp.concatenate([x_ref[k, b, 0] for k in range(_K)], axis=-1)
            tid = tid_ref[b, 0, 0, :]
            oh = (tid[:, None] == lax.broadcasted_iota(jnp.int32, (_BT, 8), 1)).astype(
                jnp.float32
            )
            te = jnp.dot(oh, tt_ref[...], preferred_element_type=jnp.float32)
            y = x + pe + te
            if pool:
                ii = lax.broadcasted_iota(jnp.int32, (_BT // 4, _BT), 0)
                jj = lax.broadcasted_iota(jnp.int32, (_BT // 4, _BT), 1)
                pmat = jnp.where(ii == jj // 4, 0.25, 0.0).astype(jnp.float32)
                y = jnp.dot(pmat, y, preferred_element_type=jnp.float32)
            g = y * gate_ref[...]
            ms = jnp.mean(g * g, axis=-1, keepdims=True)
            o_ref[b] = g * lax.rsqrt(ms + 1e-6) * w_ref[...]

    return body


def _post_body_pool_aliased(B):
    def body(x_ref, pe_ref, tid_ref, tt_ref, gate_ref, w_ref, alias_ref, o_ref):
        _post_body(True, B)(x_ref, pe_ref, tid_ref, tt_ref, gate_ref, w_ref, o_ref)

    return body


def kernel(token_ids, type_ids, hash_tables, type_table, elastic_gate, rms_weight):
    B, T = token_ids.shape
    half = T // 2
    n_out = half // 4 + half

    table_flat = hash_tables.reshape(_K * _N_BUCKETS, _D_HASH)
    tok_old = token_ids[:, :half].reshape(B * half)
    tok_rec = token_ids[:, half:].reshape(B * half)
    sc = _make_sc_gather(B * half)
    rows_rec = sc(tok_rec, table_flat)
    rows_old = sc(tok_old, table_flat)
    x_old = rows_old.reshape(_K, B, half // _BT, _BT, _D_HASH)
    x_rec = rows_rec.reshape(_K, B, half // _BT, _BT, _D_HASH)

    pe = _pos_encoding(T, _D_MODEL).astype(jnp.bfloat16)
    tid4 = type_ids.reshape(B, T // _BT, 1, _BT)
    tt_pad = jnp.zeros((8, _D_MODEL), jnp.float32).at[:_N_TYPES].set(type_table)
    gate2 = elastic_gate.reshape(1, _D_MODEL)
    w2 = rms_weight.reshape(1, _D_MODEL)

    nblk = half // _BT
    common_specs = [
        pl.BlockSpec((8, _D_MODEL), lambda i: (0, 0)),
        pl.BlockSpec((1, _D_MODEL), lambda i: (0, 0)),
        pl.BlockSpec((1, _D_MODEL), lambda i: (0, 0)),
    ]
    out_sd = jax.ShapeDtypeStruct((B, n_out, _D_MODEL), jnp.float32)

    n_pool_blk = (half // 4) // _BT
    recent = pl.pallas_call(
        _post_body(False, B),
        grid=(nblk,),
        in_specs=[
            pl.BlockSpec((_K, B, 1, _BT, _D_HASH), lambda i: (0, 0, i, 0, 0)),
            pl.BlockSpec((_BT, _D_MODEL), lambda i: (nblk + i, 0)),
            pl.BlockSpec((B, 1, 1, _BT), lambda i: (0, nblk + i, 0, 0)),
            *common_specs,
        ],
        out_specs=pl.BlockSpec((B, _BT, _D_MODEL), lambda i: (0, n_pool_blk + i, 0)),
        out_shape=out_sd,
        compiler_params=pltpu.CompilerParams(dimension_semantics=("parallel",)),
    )(x_rec, pe, tid4, tt_pad, gate2, w2)

    ctx = pl.pallas_call(
        _post_body_pool_aliased(B),
        grid=(nblk,),
        in_specs=[
            pl.BlockSpec((_K, B, 1, _BT, _D_HASH), lambda i: (0, 0, i, 0, 0)),
            pl.BlockSpec((_BT, _D_MODEL), lambda i: (i, 0)),
            pl.BlockSpec((B, 1, 1, _BT), lambda i: (0, i, 0, 0)),
            *common_specs,
            pl.BlockSpec(memory_space=pl.ANY),
        ],
        out_specs=pl.BlockSpec((B, _BT // 4, _D_MODEL), lambda i: (0, i, 0)),
        out_shape=out_sd,
        input_output_aliases={6: 0},
        compiler_params=pltpu.CompilerParams(dimension_semantics=("parallel",)),
    )(x_old, pe, tid4, tt_pad, gate2, w2, recent)

    return ctx


# speedup vs baseline: 1.0984x; 1.0984x over previous
import functools
import math

import jax
import jax.numpy as jnp
import numpy as np
from jax import lax
from jax.experimental import pallas as pl
from jax.experimental.pallas import tpu as pltpu
from jax.experimental.pallas import tpu_sc as plsc

_D_MODEL = 1024
_K = 8
_D_HASH = 128
_N_BUCKETS = 32768
_N_TYPES = 6
_PRIMES = (10007, 10009, 10037, 10039, 10061, 10067, 10069, 10079)
_OFFSETS = (11, 29, 47, 71, 101, 131, 149, 173)


def _pos_encoding(T, d):
    position = np.arange(T, dtype=np.float32)[:, None]
    div_term = np.exp(np.arange(0, d, 2, dtype=np.float32) * (-math.log(10000.0) / d))
    pe = np.zeros((T, d), dtype=np.float32)
    pe[:, 0::2] = np.sin(position * div_term)
    pe[:, 1::2] = np.cos(position * div_term)
    return jnp.asarray(pe)


@functools.lru_cache(maxsize=None)
def _make_sc_gather(n_tok, pool4=False):
    info = plsc.get_sparse_core_info()
    nc, ns = info.num_cores, info.num_subcores
    nw = nc * ns
    tok_per_w = n_tok // nw
    n_rows = n_tok * _K

    mesh = plsc.VectorSubcoreMesh(core_axis_name="c", subcore_axis_name="s")

    n_gg = tok_per_w // 128
    ng = _K * n_gg
    nb = 6
    fire_ahead = 2

    orows = 32 if pool4 else 128
    n_orows = (n_rows // 4) if pool4 else n_rows
    o_tok = (n_tok // 4) if pool4 else n_tok
    ow_base_mul = (tok_per_w // 4) if pool4 else tok_per_w

    scratch = [
        pltpu.VMEM((tok_per_w,), jnp.int32),
        pltpu.VMEM((ng, 128), jnp.int32),
        pltpu.VMEM((nb * 128, _D_HASH), jnp.float32),
        pltpu.SemaphoreType.DMA,
        pltpu.SemaphoreType.DMA,
    ]
    if pool4:
        scratch.insert(3, pltpu.VMEM((nb * 32, _D_HASH), jnp.float32))

    @functools.partial(
        pl.kernel,
        mesh=mesh,
        out_type=jax.ShapeDtypeStruct((n_orows, _D_HASH), jnp.float32),
        scratch_types=scratch,
    )
    def sc_gather(tok_hbm, table_hbm, out_hbm, tok_v, idx_v, rows_v, *rest):
        if pool4:
            pbuf_v, gsem, osem = rest
        else:
            gsem, osem = rest
        wid = lax.axis_index("s") * nc + lax.axis_index("c")
        pltpu.sync_copy(tok_hbm.at[pl.ds(wid * tok_per_w, tok_per_w)], tok_v)

        for k in range(_K):
            for gg in range(n_gg):

                def cbody(j, carry, k=k, gg=gg):
                    tokv = tok_v[pl.ds(gg * 128 + j * 16, 16)]
                    hk = ((tokv * _PRIMES[k] + _OFFSETS[k]) & (_N_BUCKETS - 1)) + k * _N_BUCKETS
                    idx_v[k * n_gg + gg, pl.ds(j * 16, 16)] = hk
                    return carry

                lax.fori_loop(0, 8, cbody, 0)

        wbase = wid * ow_base_mul

        gg_shift = n_gg.bit_length() - 1

        def obase(g):
            return (g >> gg_shift) * o_tok + wbase + (g & (n_gg - 1)) * orows

        def slot_ref(g):
            return rows_v.at[pl.ds(lax.rem(g, nb) * 128, 128)]

        def oslot_ref(g):
            if pool4:
                return pbuf_v.at[pl.ds(lax.rem(g, nb) * 32, 32)]
            return slot_ref(g)

        def body(g, carry):
            @pl.when(g < ng)
            def _():
                @pl.when(g >= nb)
                def _():
                    pltpu.make_async_copy(
                        oslot_ref(g), out_hbm.at[pl.ds(obase(g - nb), orows)], osem
                    ).wait()

                pltpu.async_copy(table_hbm.at[idx_v.at[g]], slot_ref(g), gsem)

            h = g - fire_ahead

            @pl.when(h >= 0)
            def _():
                pltpu.make_async_copy(
                    table_hbm.at[idx_v.at[h]], slot_ref(h), gsem
                ).wait()
                if pool4:
                    rbase = lax.rem(h, nb) * 128
                    pbase = lax.rem(h, nb) * 32

                    def pbody(r, carry2):
                        row = rbase + r * 4
                        prow = pbase + r
                        for cc in range(8):
                            s = pl.ds(cc * 16, 16)
                            acc = (rows_v[row, s] + rows_v[row + 1, s]) + (
                                rows_v[row + 2, s] + rows_v[row + 3, s]
                            )
                            pbuf_v[prow, s] = acc
                        return carry2

                    lax.fori_loop(0, 32, pbody, 0)
                pltpu.async_copy(
                    oslot_ref(h), out_hbm.at[pl.ds(obase(h), orows)], osem
                )

            return carry

        lax.fori_loop(0, ng + fire_ahead, body, 0)

        def drain(i, carry):
            g = ng - nb + i
            pltpu.make_async_copy(
                oslot_ref(g), out_hbm.at[pl.ds(obase(g), orows)], osem
            ).wait()
            return carry

        lax.fori_loop(0, nb, drain, 0)

    return sc_gather


_BT = 512


def _post_body(pool, B):
    def body(x_ref, pe_ref, tid_ref, tt_ref, gate_ref, w_ref, o_ref):
        pe = pe_ref[...].astype(jnp.float32)
        for b in range(B):
            x = jnp.concatenate([x_ref[k, b, 0] for k in range(_K)], axis=-1)
            tid = tid_ref[b, 0, 0, :]
            oh = (tid[:, None] == lax.broadcasted_iota(jnp.int32, (_BT, 8), 1)).astype(
                jnp.float32
            )
            te = jnp.dot(oh, tt_ref[...], preferred_element_type=jnp.float32)
            y = x + pe + te
            if pool:
                ii = lax.broadcasted_iota(jnp.int32, (_BT // 4, _BT), 0)
                jj = lax.broadcasted_iota(jnp.int32, (_BT // 4, _BT), 1)
                pmat = jnp.where(ii == jj // 4, 0.25, 0.0).astype(jnp.float32)
                y = jnp.dot(pmat, y, preferred_element_type=jnp.float32)
            g = y * gate_ref[...]
            ms = jnp.mean(g * g, axis=-1, keepdims=True)
            o_ref[b] = g * lax.rsqrt(ms + 1e-6) * w_ref[...]

    return body


def _post_body_pool_aliased(B):
    def body(x_ref, pe_ref, tid_ref, tt_ref, gate_ref, w_ref, alias_ref, o_ref):
        pe = pe_ref[...].astype(jnp.float32)
        ii = lax.broadcasted_iota(jnp.int32, (_BT // 4, _BT), 0)
        jj = lax.broadcasted_iota(jnp.int32, (_BT // 4, _BT), 1)
        pmat = jnp.where(ii == jj // 4, 0.25, 0.0).astype(jnp.float32)
        for b in range(B):
            xp = jnp.concatenate([x_ref[k, b, 0] for k in range(_K)], axis=-1) * 0.25
            tid = tid_ref[b, 0, 0, :]
            oh = (tid[:, None] == lax.broadcasted_iota(jnp.int32, (_BT, 8), 1)).astype(
                jnp.float32
            )
            poh = jnp.dot(pmat, oh, preferred_element_type=jnp.float32)
            te = jnp.dot(poh, tt_ref[...], preferred_element_type=jnp.float32)
            g = (xp + pe + te) * gate_ref[...]
            ms = jnp.mean(g * g, axis=-1, keepdims=True)
            o_ref[b] = g * lax.rsqrt(ms + 1e-6) * w_ref[...]

    return body


def kernel(token_ids, type_ids, hash_tables, type_table, elastic_gate, rms_weight):
    B, T = token_ids.shape
    half = T // 2
    n_out = half // 4 + half

    table_flat = hash_tables.reshape(_K * _N_BUCKETS, _D_HASH)
    tok_old = token_ids[:, :half].reshape(B * half)
    tok_rec = token_ids[:, half:].reshape(B * half)
    rows_rec = _make_sc_gather(B * half)(tok_rec, table_flat)
    rows_old = _make_sc_gather(B * half, True)(tok_old, table_flat)
    x_rec = rows_rec.reshape(_K, B, half // _BT, _BT, _D_HASH)
    qtr = _BT // 4
    x_old = rows_old.reshape(_K, B, (half // 4) // qtr, qtr, _D_HASH)

    pe_full = _pos_encoding(T, _D_MODEL)
    pe = pe_full.astype(jnp.bfloat16)
    pe_p = (
        pe_full[:half].reshape(half // 4, 4, _D_MODEL).mean(axis=1).astype(jnp.bfloat16)
    )
    tid4 = type_ids.reshape(B, T // _BT, 1, _BT)
    tt_pad = jnp.zeros((8, _D_MODEL), jnp.float32).at[:_N_TYPES].set(type_table)
    gate2 = elastic_gate.reshape(1, _D_MODEL)
    w2 = rms_weight.reshape(1, _D_MODEL)

    nblk = half // _BT
    common_specs = [
        pl.BlockSpec((8, _D_MODEL), lambda i: (0, 0)),
        pl.BlockSpec((1, _D_MODEL), lambda i: (0, 0)),
        pl.BlockSpec((1, _D_MODEL), lambda i: (0, 0)),
    ]
    out_sd = jax.ShapeDtypeStruct((B, n_out, _D_MODEL), jnp.float32)

    n_pool_blk = (half // 4) // _BT
    recent = pl.pallas_call(
        _post_body(False, B),
        grid=(nblk,),
        in_specs=[
            pl.BlockSpec((_K, B, 1, _BT, _D_HASH), lambda i: (0, 0, i, 0, 0)),
            pl.BlockSpec((_BT, _D_MODEL), lambda i: (nblk + i, 0)),
            pl.BlockSpec((B, 1, 1, _BT), lambda i: (0, nblk + i, 0, 0)),
            *common_specs,
        ],
        out_specs=pl.BlockSpec((B, _BT, _D_MODEL), lambda i: (0, n_pool_blk + i, 0)),
        out_shape=out_sd,
        compiler_params=pltpu.CompilerParams(dimension_semantics=("parallel",)),
    )(x_rec, pe, tid4, tt_pad, gate2, w2)

    ctx = pl.pallas_call(
        _post_body_pool_aliased(B),
        grid=(nblk,),
        in_specs=[
            pl.BlockSpec((_K, B, 1, qtr, _D_HASH), lambda i: (0, 0, i, 0, 0)),
            pl.BlockSpec((qtr, _D_MODEL), lambda i: (i, 0)),
            pl.BlockSpec((B, 1, 1, _BT), lambda i: (0, i, 0, 0)),
            *common_specs,
            pl.BlockSpec(memory_space=pl.ANY),
        ],
        out_specs=pl.BlockSpec((B, _BT // 4, _D_MODEL), lambda i: (0, i, 0)),
        out_shape=out_sd,
        input_output_aliases={6: 0},
        compiler_params=pltpu.CompilerParams(dimension_semantics=("parallel",)),
    )(x_old, pe_p, tid4, tt_pad, gate2, w2, recent)

    return ctx


# speedup vs baseline: 1.1223x; 1.0218x over previous
import functools
import math

import jax
import jax.numpy as jnp
import numpy as np
from jax import lax
from jax.experimental import pallas as pl
from jax.experimental.pallas import tpu as pltpu
from jax.experimental.pallas import tpu_sc as plsc

_D_MODEL = 1024
_K = 8
_D_HASH = 128
_N_BUCKETS = 32768
_N_TYPES = 6
_PRIMES = (10007, 10009, 10037, 10039, 10061, 10067, 10069, 10079)
_OFFSETS = (11, 29, 47, 71, 101, 131, 149, 173)


def _pos_encoding(T, d):
    position = np.arange(T, dtype=np.float32)[:, None]
    div_term = np.exp(np.arange(0, d, 2, dtype=np.float32) * (-math.log(10000.0) / d))
    pe = np.zeros((T, d), dtype=np.float32)
    pe[:, 0::2] = np.sin(position * div_term)
    pe[:, 1::2] = np.cos(position * div_term)
    return jnp.asarray(pe)


@functools.lru_cache(maxsize=None)
def _make_sc_gather(n_tok, pool4=False):
    info = plsc.get_sparse_core_info()
    nc, ns = info.num_cores, info.num_subcores
    nw = nc * ns
    tok_per_w = n_tok // nw
    n_rows = n_tok * _K

    mesh = plsc.VectorSubcoreMesh(core_axis_name="c", subcore_axis_name="s")

    n_gg = tok_per_w // 128
    ng = _K * n_gg
    nb = 6
    fire_ahead = 2

    orows = 32 if pool4 else 128
    n_orows = (n_rows // 4) if pool4 else n_rows
    o_tok = (n_tok // 4) if pool4 else n_tok
    ow_base_mul = (tok_per_w // 4) if pool4 else tok_per_w

    scratch = [
        pltpu.VMEM((tok_per_w,), jnp.int32),
        pltpu.VMEM((ng, 128), jnp.int32),
        pltpu.VMEM((nb * 128, _D_HASH), jnp.float32),
        pltpu.SemaphoreType.DMA,
        pltpu.SemaphoreType.DMA,
    ]
    if pool4:
        scratch.insert(3, pltpu.VMEM((nb * 32, _D_HASH), jnp.float32))

    @functools.partial(
        pl.kernel,
        mesh=mesh,
        out_type=jax.ShapeDtypeStruct((n_orows, _D_HASH), jnp.float32),
        scratch_types=scratch,
    )
    def sc_gather(tok_hbm, table_hbm, out_hbm, tok_v, idx_v, rows_v, *rest):
        if pool4:
            pbuf_v, gsem, osem = rest
        else:
            gsem, osem = rest
        wid = lax.axis_index("s") * nc + lax.axis_index("c")
        pltpu.sync_copy(tok_hbm.at[pl.ds(wid * tok_per_w, tok_per_w)], tok_v)

        for k in range(_K):
            for gg in range(n_gg):

                def cbody(j, carry, k=k, gg=gg):
                    tokv = tok_v[pl.ds(gg * 128 + j * 16, 16)]
                    hk = ((tokv * _PRIMES[k] + _OFFSETS[k]) & (_N_BUCKETS - 1)) + k * _N_BUCKETS
                    idx_v[k * n_gg + gg, pl.ds(j * 16, 16)] = hk
                    return carry

                lax.fori_loop(0, 8, cbody, 0)

        wbase = wid * ow_base_mul

        gg_shift = n_gg.bit_length() - 1

        def obase(g):
            return (g >> gg_shift) * o_tok + wbase + (g & (n_gg - 1)) * orows

        def slot_ref(g):
            return rows_v.at[pl.ds(lax.rem(g, nb) * 128, 128)]

        def oslot_ref(g):
            if pool4:
                return pbuf_v.at[pl.ds(lax.rem(g, nb) * 32, 32)]
            return slot_ref(g)

        def body(g, carry):
            @pl.when(g < ng)
            def _():
                @pl.when(g >= nb)
                def _():
                    pltpu.make_async_copy(
                        oslot_ref(g), out_hbm.at[pl.ds(obase(g - nb), orows)], osem
                    ).wait()

                pltpu.async_copy(table_hbm.at[idx_v.at[g]], slot_ref(g), gsem)

            h = g - fire_ahead

            @pl.when(h >= 0)
            def _():
                pltpu.make_async_copy(
                    table_hbm.at[idx_v.at[h]], slot_ref(h), gsem
                ).wait()
                if pool4:
                    rbase = lax.rem(h, nb) * 128
                    pbase = lax.rem(h, nb) * 32

                    def pbody(r, carry2):
                        row = rbase + r * 4
                        prow = pbase + r
                        for cc in range(8):
                            s = pl.ds(cc * 16, 16)
                            acc = (rows_v[row, s] + rows_v[row + 1, s]) + (
                                rows_v[row + 2, s] + rows_v[row + 3, s]
                            )
                            pbuf_v[prow, s] = acc
                        return carry2

                    lax.fori_loop(0, 32, pbody, 0)
                pltpu.async_copy(
                    oslot_ref(h), out_hbm.at[pl.ds(obase(h), orows)], osem
                )

            return carry

        lax.fori_loop(0, ng + fire_ahead, body, 0)

        def drain(i, carry):
            g = ng - nb + i
            pltpu.make_async_copy(
                oslot_ref(g), out_hbm.at[pl.ds(obase(g), orows)], osem
            ).wait()
            return carry

        lax.fori_loop(0, nb, drain, 0)

    return sc_gather


_BT = 1024


def _post_body(pool, B):
    def body(x_ref, pe_ref, tid_ref, tt_ref, gate_ref, w_ref, o_ref):
        pe = pe_ref[...].astype(jnp.float32)
        for b in range(B):
            x = jnp.concatenate([x_ref[k, b, 0] for k in range(_K)], axis=-1)
            tid = tid_ref[b, 0, 0, :]
            oh = (tid[:, None] == lax.broadcasted_iota(jnp.int32, (_BT, 8), 1)).astype(
                jnp.float32
            )
            te = jnp.dot(oh, tt_ref[...], preferred_element_type=jnp.float32)
            y = x + pe + te
            if pool:
                ii = lax.broadcasted_iota(jnp.int32, (_BT // 4, _BT), 0)
                jj = lax.broadcasted_iota(jnp.int32, (_BT // 4, _BT), 1)
                pmat = jnp.where(ii == jj // 4, 0.25, 0.0).astype(jnp.float32)
                y = jnp.dot(pmat, y, preferred_element_type=jnp.float32)
            g = y * gate_ref[...]
            ms = jnp.mean(g * g, axis=-1, keepdims=True)
            o_ref[b] = g * lax.rsqrt(ms + 1e-6) * w_ref[...]

    return body


def _post_body_pool_aliased(B):
    def body(x_ref, pe_ref, tid_ref, tt_ref, gate_ref, w_ref, alias_ref, o_ref):
        pe = pe_ref[...].astype(jnp.float32)
        ii = lax.broadcasted_iota(jnp.int32, (_BT // 4, _BT), 0)
        jj = lax.broadcasted_iota(jnp.int32, (_BT // 4, _BT), 1)
        pmat = jnp.where(ii == jj // 4, 0.25, 0.0).astype(jnp.float32)
        for b in range(B):
            xp = jnp.concatenate([x_ref[k, b, 0] for k in range(_K)], axis=-1) * 0.25
            tid = tid_ref[b, 0, 0, :]
            oh = (tid[:, None] == lax.broadcasted_iota(jnp.int32, (_BT, 8), 1)).astype(
                jnp.float32
            )
            poh = jnp.dot(pmat, oh, preferred_element_type=jnp.float32)
            te = jnp.dot(poh, tt_ref[...], preferred_element_type=jnp.float32)
            g = (xp + pe + te) * gate_ref[...]
            ms = jnp.mean(g * g, axis=-1, keepdims=True)
            o_ref[b] = g * lax.rsqrt(ms + 1e-6) * w_ref[...]

    return body


def kernel(token_ids, type_ids, hash_tables, type_table, elastic_gate, rms_weight):
    B, T = token_ids.shape
    half = T // 2
    n_out = half // 4 + half

    table_flat = hash_tables.reshape(_K * _N_BUCKETS, _D_HASH)
    tok_old = token_ids[:, :half].reshape(B * half)
    tok_rec = token_ids[:, half:].reshape(B * half)
    rows_rec = _make_sc_gather(B * half)(tok_rec, table_flat)
    rows_old = _make_sc_gather(B * half, True)(tok_old, table_flat)
    x_rec = rows_rec.reshape(_K, B, half // _BT, _BT, _D_HASH)
    qtr = _BT // 4
    x_old = rows_old.reshape(_K, B, (half // 4) // qtr, qtr, _D_HASH)

    pe_full = _pos_encoding(T, _D_MODEL)
    pe = pe_full.astype(jnp.bfloat16)
    pe_p = (
        pe_full[:half].reshape(half // 4, 4, _D_MODEL).mean(axis=1).astype(jnp.bfloat16)
    )
    tid4 = type_ids.reshape(B, T // _BT, 1, _BT)
    tt_pad = jnp.zeros((8, _D_MODEL), jnp.float32).at[:_N_TYPES].set(type_table)
    gate2 = elastic_gate.reshape(1, _D_MODEL)
    w2 = rms_weight.reshape(1, _D_MODEL)

    nblk = half // _BT
    common_specs = [
        pl.BlockSpec((8, _D_MODEL), lambda i: (0, 0)),
        pl.BlockSpec((1, _D_MODEL), lambda i: (0, 0)),
        pl.BlockSpec((1, _D_MODEL), lambda i: (0, 0)),
    ]
    out_sd = jax.ShapeDtypeStruct((B, n_out, _D_MODEL), jnp.float32)

    n_pool_blk = (half // 4) // _BT
    recent = pl.pallas_call(
        _post_body(False, B),
        grid=(nblk,),
        in_specs=[
            pl.BlockSpec((_K, B, 1, _BT, _D_HASH), lambda i: (0, 0, i, 0, 0)),
            pl.BlockSpec((_BT, _D_MODEL), lambda i: (nblk + i, 0)),
            pl.BlockSpec((B, 1, 1, _BT), lambda i: (0, nblk + i, 0, 0)),
            *common_specs,
        ],
        out_specs=pl.BlockSpec((B, _BT, _D_MODEL), lambda i: (0, n_pool_blk + i, 0)),
        out_shape=out_sd,
        compiler_params=pltpu.CompilerParams(dimension_semantics=("parallel",)),
    )(x_rec, pe, tid4, tt_pad, gate2, w2)

    ctx = pl.pallas_call(
        _post_body_pool_aliased(B),
        grid=(nblk,),
        in_specs=[
            pl.BlockSpec((_K, B, 1, qtr, _D_HASH), lambda i: (0, 0, i, 0, 0)),
            pl.BlockSpec((qtr, _D_MODEL), lambda i: (i, 0)),
            pl.BlockSpec((B, 1, 1, _BT), lambda i: (0, i, 0, 0)),
            *common_specs,
            pl.BlockSpec(memory_space=pl.ANY),
        ],
        out_specs=pl.BlockSpec((B, _BT // 4, _D_MODEL), lambda i: (0, i, 0)),
        out_shape=out_sd,
        input_output_aliases={6: 0},
        compiler_params=pltpu.CompilerParams(dimension_semantics=("parallel",)),
    )(x_old, pe_p, tid4, tt_pad, gate2, w2, recent)

    return ctx
